# baseline (speedup 1.0000x reference)
import functools

import jax
import jax.numpy as jnp
from jax import lax
from jax.experimental import pallas as pl
from jax.experimental.pallas import tpu as pltpu
from jax.experimental.pallas import tpu_sc as plsc

N = 10000
E = 320000
F = 128
NUM_OUT = 16
NC, NS = 2, 16
NW = NC * NS
EPT = E // NW
CH = 128
NCH_E = 80
NCH_A = NCH_E + 4
EPT_PAD = NCH_A * CH
IRING = 8
DUMMY = N
N_PAD = 10240
SLAB = N_PAD // NS


def _sc_mesh():
    return plsc.VectorSubcoreMesh(
        core_axis_name="c", subcore_axis_name="s", num_cores=NC, num_subcores=NS
    )


def _deg_body(dst_hbm, zeros1_hbm, deg_hbm, dstv, onesv, degsh):
    cid = lax.axis_index("c")
    sid = lax.axis_index("s")
    wid = cid * NS + sid
    pltpu.sync_copy(zeros1_hbm.at[pl.ds(sid * SLAB, SLAB)],
                    degsh.at[pl.ds(sid * SLAB, SLAB)])
    pltpu.sync_copy(dst_hbm.at[wid], dstv)
    for i in range(CH // 16):
        onesv[pl.ds(i * 16, 16)] = jnp.ones((16,), jnp.float32)
    plsc.subcore_barrier()

    def step(j, carry):
        pltpu.sync_copy(onesv, degsh.at[dstv.at[j]], add=True)
        return carry

    lax.fori_loop(0, NCH_E, step, 0)
    plsc.subcore_barrier()
    pltpu.sync_copy(degsh.at[pl.ds(sid * SLAB, SLAB)],
                    deg_hbm.at[cid, pl.ds(sid * SLAB, SLAB)])


def _sc_degree(dst3, zeros1):
    return pl.kernel(
        _deg_body,
        out_type=jax.ShapeDtypeStruct((NC, N_PAD), jnp.float32),
        mesh=_sc_mesh(),
        scratch_types=[
            pltpu.VMEM((NCH_A, CH), jnp.int32),
            pltpu.VMEM((CH,), jnp.float32),
            pltpu.VMEM_SHARED((N_PAD,), jnp.float32),
        ],
    )(dst3, zeros1)


def _agg_body(src_hbm, dst_hbm, yp_hbm, norm_hbm, zeros1_hbm, zeros2_hbm,
              raw_hbm, t_hbm, srcv, dstv, rows0, rows1, nb0, nb1,
              rawsh, tsh, semi, semr0, semr1, semng0, semng1, semns0, semns1):
    cid = lax.axis_index("c")
    sid = lax.axis_index("s")
    wid = cid * NS + sid
    pltpu.sync_copy(zeros2_hbm.at[pl.ds(sid * SLAB, SLAB)],
                    rawsh.at[pl.ds(sid * SLAB, SLAB)])
    pltpu.sync_copy(zeros1_hbm.at[pl.ds(sid * SLAB, SLAB)],
                    tsh.at[pl.ds(sid * SLAB, SLAB)])
    pltpu.sync_copy(src_hbm.at[wid, 0], srcv.at[0])
    pltpu.sync_copy(dst_hbm.at[wid, 0], dstv.at[0])
    pltpu.sync_copy(src_hbm.at[wid, 1], srcv.at[1])
    pltpu.sync_copy(dst_hbm.at[wid, 1], dstv.at[1])
    pltpu.async_copy(src_hbm.at[wid, 2], srcv.at[2], semi)
    pltpu.async_copy(dst_hbm.at[wid, 2], dstv.at[2], semi)
    pltpu.async_copy(src_hbm.at[wid, 3], srcv.at[3], semi)
    pltpu.async_copy(dst_hbm.at[wid, 3], dstv.at[3], semi)
    plsc.subcore_barrier()
    pltpu.async_copy(yp_hbm.at[srcv.at[0]], rows0, semr0)
    pltpu.async_copy(yp_hbm.at[srcv.at[1]], rows1, semr1)
    pltpu.async_copy(norm_hbm.at[dstv.at[0]], nb0, semng0)
    pltpu.async_copy(norm_hbm.at[dstv.at[1]], nb1, semng1)

    def step(p, carry):
        a = 2 * p
        b = a + 1
        sa = lax.rem(a, IRING)
        sb = lax.rem(b, IRING)
        s2a = lax.rem(a + 2, IRING)
        s2b = lax.rem(b + 2, IRING)
        s4a = lax.rem(a + 4, IRING)
        s4b = lax.rem(b + 4, IRING)
        for _ in range(4):
            pltpu.make_async_copy(src_hbm.at[wid, 0], srcv.at[s2a], semi).wait()
        pltpu.async_copy(src_hbm.at[wid, a + 4], srcv.at[s4a], semi)
        pltpu.async_copy(dst_hbm.at[wid, a + 4], dstv.at[s4a], semi)
        pltpu.async_copy(src_hbm.at[wid, b + 4], srcv.at[s4b], semi)
        pltpu.async_copy(dst_hbm.at[wid, b + 4], dstv.at[s4b], semi)
        pltpu.make_async_copy(yp_hbm.at[srcv.at[sa]], rows0, semr0).wait()
        pltpu.sync_copy(rows0, rawsh.at[dstv.at[sa]], add=True)
        pltpu.make_async_copy(norm_hbm.at[dstv.at[sa]], nb0, semng0).wait()
        pltpu.async_copy(nb0, tsh.at[srcv.at[sa]], semns0, add=True)
        pltpu.make_async_copy(yp_hbm.at[srcv.at[sb]], rows1, semr1).wait()
        pltpu.sync_copy(rows1, rawsh.at[dstv.at[sb]], add=True)
        pltpu.make_async_copy(norm_hbm.at[dstv.at[sb]], nb1, semng1).wait()
        pltpu.async_copy(nb1, tsh.at[srcv.at[sb]], semns1, add=True)
        pltpu.async_copy(yp_hbm.at[srcv.at[s2a]], rows0, semr0)
        pltpu.async_copy(yp_hbm.at[srcv.at[s2b]], rows1, semr1)
        pltpu.make_async_copy(nb0, tsh.at[srcv.at[sa]], semns0).wait()
        pltpu.async_copy(norm_hbm.at[dstv.at[s2a]], nb0, semng0)
        pltpu.make_async_copy(nb1, tsh.at[srcv.at[sb]], semns1).wait()
        pltpu.async_copy(norm_hbm.at[dstv.at[s2b]], nb1, semng1)
        return carry

    lax.fori_loop(0, NCH_E // 2, step, 0)
    for _ in range(4):
        pltpu.make_async_copy(src_hbm.at[wid, 0], srcv.at[0], semi).wait()
    pltpu.make_async_copy(yp_hbm.at[srcv.at[0]], rows0, semr0).wait()
    pltpu.make_async_copy(yp_hbm.at[srcv.at[1]], rows1, semr1).wait()
    pltpu.make_async_copy(norm_hbm.at[dstv.at[0]], nb0, semng0).wait()
    pltpu.make_async_copy(norm_hbm.at[dstv.at[1]], nb1, semng1).wait()
    plsc.subcore_barrier()
    pltpu.sync_copy(rawsh.at[pl.ds(sid * SLAB, SLAB)],
                    raw_hbm.at[cid, pl.ds(sid * SLAB, SLAB)])
    pltpu.sync_copy(tsh.at[pl.ds(sid * SLAB, SLAB)],
                    t_hbm.at[cid, pl.ds(sid * SLAB, SLAB)])


def _sc_aggregate(src3, dst3, yp, norm, zeros1, zeros2):
    return pl.kernel(
        _agg_body,
        out_type=(
            jax.ShapeDtypeStruct((NC, N_PAD, F), jnp.float32),
            jax.ShapeDtypeStruct((NC, N_PAD), jnp.float32),
        ),
        mesh=_sc_mesh(),
        scratch_types=[
            pltpu.VMEM((IRING, CH), jnp.int32),
            pltpu.VMEM((IRING, CH), jnp.int32),
            pltpu.VMEM((CH, F), jnp.float32),
            pltpu.VMEM((CH, F), jnp.float32),
            pltpu.VMEM((CH,), jnp.float32),
            pltpu.VMEM((CH,), jnp.float32),
            pltpu.VMEM_SHARED((N_PAD, F), jnp.float32),
            pltpu.VMEM_SHARED((N_PAD,), jnp.float32),
            pltpu.SemaphoreType.DMA,
            pltpu.SemaphoreType.DMA,
            pltpu.SemaphoreType.DMA,
            pltpu.SemaphoreType.DMA,
            pltpu.SemaphoreType.DMA,
            pltpu.SemaphoreType.DMA,
            pltpu.SemaphoreType.DMA,
        ],
    )(src3, dst3, yp, norm, zeros1, zeros2)


BLK = 1024


def _prep_body(dega_ref, degb_ref, x_ref, w1_ref, yp_ref, norm_ref):
    deg = dega_ref[...] + degb_ref[...]
    norm = jnp.where(deg > 0.0, lax.rsqrt(deg), 0.0)
    y = jnp.dot(x_ref[...], w1_ref[...], preferred_element_type=jnp.float32)
    yp_ref[...] = y * norm
    norm_ref[...] = norm


def _tc_prep(dega, degb, x_pad, W1):
    grid = N_PAD // BLK
    return pl.pallas_call(
        _prep_body,
        grid=(grid,),
        in_specs=[
            pl.BlockSpec((BLK, 1), lambda i: (i, 0)),
            pl.BlockSpec((BLK, 1), lambda i: (i, 0)),
            pl.BlockSpec((BLK, F), lambda i: (i, 0)),
            pl.BlockSpec((F, F), lambda i: (0, 0)),
        ],
        out_specs=[
            pl.BlockSpec((BLK, F), lambda i: (i, 0)),
            pl.BlockSpec((BLK, 1), lambda i: (i, 0)),
        ],
        out_shape=[
            jax.ShapeDtypeStruct((N_PAD, F), jnp.float32),
            jax.ShapeDtypeStruct((N_PAD, 1), jnp.float32),
        ],
    )(dega, degb, x_pad, W1)


def _final_body(rawa_ref, rawb_ref, ta_ref, tb_ref, norm_ref, b1_ref,
                w2_ref, b2_ref, out_ref, acc_ref):
    i = pl.program_id(0)
    norm = norm_ref[...]
    h1 = jnp.maximum(norm * (rawa_ref[...] + rawb_ref[...]) + b1_ref[...], 0.0)
    s = norm * (ta_ref[...] + tb_ref[...])
    gid = i * BLK + lax.broadcasted_iota(jnp.int32, (BLK, 1), 0)
    s = jnp.where(gid < N, s, 0.0)
    contrib = jnp.sum(s * h1, axis=0, keepdims=True)

    @pl.when(i == 0)
    def _():
        acc_ref[...] = jnp.zeros_like(acc_ref)

    acc_ref[...] += contrib

    @pl.when(i == pl.num_programs(0) - 1)
    def _():
        v = acc_ref[...] * (1.0 / N)
        out_ref[...] = (
            jnp.dot(v, w2_ref[...], preferred_element_type=jnp.float32)
            + b2_ref[...]
        )


def _tc_final(rawa, rawb, ta, tb, norm, b1r, W2p, b2p):
    grid = N_PAD // BLK
    return pl.pallas_call(
        _final_body,
        grid=(grid,),
        in_specs=[
            pl.BlockSpec((BLK, F), lambda i: (i, 0)),
            pl.BlockSpec((BLK, F), lambda i: (i, 0)),
            pl.BlockSpec((BLK, 1), lambda i: (i, 0)),
            pl.BlockSpec((BLK, 1), lambda i: (i, 0)),
            pl.BlockSpec((BLK, 1), lambda i: (i, 0)),
            pl.BlockSpec((1, F), lambda i: (0, 0)),
            pl.BlockSpec((F, F), lambda i: (0, 0)),
            pl.BlockSpec((1, F), lambda i: (0, 0)),
        ],
        out_specs=pl.BlockSpec((1, F), lambda i: (0, 0)),
        out_shape=jax.ShapeDtypeStruct((1, F), jnp.float32),
        scratch_shapes=[pltpu.VMEM((1, F), jnp.float32)],
    )(rawa, rawb, ta, tb, norm, b1r, W2p, b2p)


def kernel(x, edge_index, W1, b1, W2, b2):
    src = edge_index[0].astype(jnp.int32)
    dst = edge_index[1].astype(jnp.int32)
    n_pad_edges = EPT_PAD - EPT
    pad_block = DUMMY + (
        (jnp.arange(n_pad_edges, dtype=jnp.int32)[None, :]
         + 97 * jnp.arange(NW, dtype=jnp.int32)[:, None]) % (N_PAD - N))
    src3 = jnp.concatenate([src.reshape(NW, EPT), pad_block], axis=1)
    src3 = src3.reshape(NW, NCH_A, CH)
    dst3 = jnp.concatenate([dst.reshape(NW, EPT), pad_block], axis=1)
    dst3 = dst3.reshape(NW, NCH_A, CH)
    x_pad = jnp.pad(x, ((0, N_PAD - N), (0, 0)))
    zeros1 = jnp.zeros((N_PAD,), jnp.float32)
    zeros2 = jnp.zeros((N_PAD, F), jnp.float32)

    deg = _sc_degree(dst3, zeros1)
    dega = deg[0].reshape(N_PAD, 1)
    degb = deg[1].reshape(N_PAD, 1)

    yp, norm_col = _tc_prep(dega, degb, x_pad, W1)
    norm_flat = norm_col.reshape(N_PAD)

    raw, t = _sc_aggregate(src3, dst3, yp, norm_flat, zeros1, zeros2)

    W2p = jnp.pad(W2, ((0, 0), (0, F - W2.shape[1])))
    b2p = jnp.pad(b2, (0, F - b2.shape[0])).reshape(1, F)
    out = _tc_final(raw[0], raw[1], t[0].reshape(N_PAD, 1),
                    t[1].reshape(N_PAD, 1), norm_col, b1.reshape(1, F),
                    W2p, b2p)
    return out[0, :NUM_OUT]


# speedup vs baseline: 1.1823x; 1.1823x over previous
import functools

import jax
import jax.numpy as jnp
from jax import lax
from jax.experimental import pallas as pl
from jax.experimental.pallas import tpu as pltpu
from jax.experimental.pallas import tpu_sc as plsc

N = 10000
E = 320000
F = 128
NUM_OUT = 16
NC, NS = 2, 16
NW = NC * NS
EPT = E // NW
CH = 96
NCH_E = 105
NCH_A = NCH_E + 4
EPT_PAD = NCH_A * CH
IRING = 8
DUMMY = N
N_PAD = 10240
SLAB = N_PAD // NS


def _sc_mesh():
    return plsc.VectorSubcoreMesh(
        core_axis_name="c", subcore_axis_name="s", num_cores=NC, num_subcores=NS
    )


def _deg_body(dst_hbm, zeros1_hbm, deg_hbm, dstv, onesv, degsh):
    cid = lax.axis_index("c")
    sid = lax.axis_index("s")
    wid = cid * NS + sid
    pltpu.sync_copy(zeros1_hbm.at[pl.ds(sid * SLAB, SLAB)],
                    degsh.at[pl.ds(sid * SLAB, SLAB)])
    pltpu.sync_copy(dst_hbm.at[wid], dstv)
    for i in range(CH // 16):
        onesv[pl.ds(i * 16, 16)] = jnp.ones((16,), jnp.float32)
    plsc.subcore_barrier()

    def step(j, carry):
        pltpu.sync_copy(onesv, degsh.at[dstv.at[j]], add=True)
        return carry

    lax.fori_loop(0, NCH_E, step, 0)
    plsc.subcore_barrier()
    pltpu.sync_copy(degsh.at[pl.ds(sid * SLAB, SLAB)],
                    deg_hbm.at[cid, pl.ds(sid * SLAB, SLAB)])


def _sc_degree(dst3, zeros1):
    return pl.kernel(
        _deg_body,
        out_type=jax.ShapeDtypeStruct((NC, N_PAD), jnp.float32),
        mesh=_sc_mesh(),
        scratch_types=[
            pltpu.VMEM((NCH_A, CH), jnp.int32),
            pltpu.VMEM((CH,), jnp.float32),
            pltpu.VMEM_SHARED((N_PAD,), jnp.float32),
        ],
    )(dst3, zeros1)


def _agg_body(src_hbm, dst_hbm, yp_hbm, norm_hbm, zeros1_hbm, zeros2_hbm,
              raw_hbm, t_hbm, srcv, dstv, rows0, rows1, rows2, nb0, nb1, nb2,
              rawsh, tsh, semi, semr0, semr1, semr2, semng0, semng1, semng2,
              semsr0, semsr1, semsr2, semsn0, semsn1, semsn2):
    cid = lax.axis_index("c")
    sid = lax.axis_index("s")
    wid = cid * NS + sid
    rows = (rows0, rows1, rows2)
    nb = (nb0, nb1, nb2)
    semr = (semr0, semr1, semr2)
    semng = (semng0, semng1, semng2)
    semsr = (semsr0, semsr1, semsr2)
    semsn = (semsn0, semsn1, semsn2)
    pltpu.sync_copy(zeros2_hbm.at[pl.ds(sid * SLAB, SLAB)],
                    rawsh.at[pl.ds(sid * SLAB, SLAB)])
    pltpu.sync_copy(zeros1_hbm.at[pl.ds(sid * SLAB, SLAB)],
                    tsh.at[pl.ds(sid * SLAB, SLAB)])
    pltpu.sync_copy(src_hbm.at[wid, 0], srcv.at[0])
    pltpu.sync_copy(dst_hbm.at[wid, 0], dstv.at[0])
    pltpu.sync_copy(src_hbm.at[wid, 1], srcv.at[1])
    pltpu.sync_copy(dst_hbm.at[wid, 1], dstv.at[1])
    pltpu.async_copy(src_hbm.at[wid, 2], srcv.at[2], semi)
    pltpu.async_copy(dst_hbm.at[wid, 2], dstv.at[2], semi)
    pltpu.async_copy(src_hbm.at[wid, 3], srcv.at[3], semi)
    pltpu.async_copy(dst_hbm.at[wid, 3], dstv.at[3], semi)
    plsc.subcore_barrier()
    pltpu.async_copy(yp_hbm.at[srcv.at[0]], rows0, semr0)
    pltpu.async_copy(norm_hbm.at[dstv.at[0]], nb0, semng0)
    pltpu.async_copy(yp_hbm.at[srcv.at[1]], rows1, semr1)
    pltpu.async_copy(norm_hbm.at[dstv.at[1]], nb1, semng1)

    def step(t_it, carry):
        for k in range(3):
            j = 3 * t_it + k
            km = (k + 2) % 3
            sj = lax.rem(j, IRING)
            s2 = lax.rem(j + 2, IRING)
            s4 = lax.rem(j + 4, IRING)
            pltpu.make_async_copy(yp_hbm.at[srcv.at[sj]], rows[k],
                                  semr[k]).wait()
            pltpu.async_copy(rows[k], rawsh.at[dstv.at[sj]], semsr[k],
                             add=True)
            pltpu.make_async_copy(norm_hbm.at[dstv.at[sj]], nb[k],
                                  semng[k]).wait()
            pltpu.async_copy(nb[k], tsh.at[srcv.at[sj]], semsn[k], add=True)

            @pl.when(j > 0)
            def _():
                pltpu.make_async_copy(rows[km], rawsh.at[dstv.at[sj]],
                                      semsr[km]).wait()
                pltpu.make_async_copy(nb[km], tsh.at[srcv.at[sj]],
                                      semsn[km]).wait()

            pltpu.async_copy(src_hbm.at[wid, j + 4], srcv.at[s4], semi)
            pltpu.async_copy(dst_hbm.at[wid, j + 4], dstv.at[s4], semi)
            for _ in range(2):
                pltpu.make_async_copy(src_hbm.at[wid, 0], srcv.at[s2],
                                      semi).wait()
            pltpu.async_copy(yp_hbm.at[srcv.at[s2]], rows[km], semr[km])
            pltpu.async_copy(norm_hbm.at[dstv.at[s2]], nb[km], semng[km])
        return carry

    lax.fori_loop(0, NCH_E // 3, step, 0)
    pltpu.make_async_copy(rows2, rawsh.at[dstv.at[0]], semsr2).wait()
    pltpu.make_async_copy(nb2, tsh.at[srcv.at[0]], semsn2).wait()
    pltpu.make_async_copy(yp_hbm.at[srcv.at[0]], rows0, semr0).wait()
    pltpu.make_async_copy(norm_hbm.at[dstv.at[0]], nb0, semng0).wait()
    pltpu.make_async_copy(yp_hbm.at[srcv.at[1]], rows1, semr1).wait()
    pltpu.make_async_copy(norm_hbm.at[dstv.at[1]], nb1, semng1).wait()
    for _ in range(4):
        pltpu.make_async_copy(src_hbm.at[wid, 0], srcv.at[0], semi).wait()
    plsc.subcore_barrier()
    pltpu.sync_copy(rawsh.at[pl.ds(sid * SLAB, SLAB)],
                    raw_hbm.at[cid, pl.ds(sid * SLAB, SLAB)])
    pltpu.sync_copy(tsh.at[pl.ds(sid * SLAB, SLAB)],
                    t_hbm.at[cid, pl.ds(sid * SLAB, SLAB)])


def _sc_aggregate(src3, dst3, yp, norm, zeros1, zeros2):
    return pl.kernel(
        _agg_body,
        out_type=(
            jax.ShapeDtypeStruct((NC, N_PAD, F), jnp.float32),
            jax.ShapeDtypeStruct((NC, N_PAD), jnp.float32),
        ),
        mesh=_sc_mesh(),
        scratch_types=[
            pltpu.VMEM((IRING, CH), jnp.int32),
            pltpu.VMEM((IRING, CH), jnp.int32),
            pltpu.VMEM((CH, F), jnp.float32),
            pltpu.VMEM((CH, F), jnp.float32),
            pltpu.VMEM((CH, F), jnp.float32),
            pltpu.VMEM((CH,), jnp.float32),
            pltpu.VMEM((CH,), jnp.float32),
            pltpu.VMEM((CH,), jnp.float32),
            pltpu.VMEM_SHARED((N_PAD, F), jnp.float32),
            pltpu.VMEM_SHARED((N_PAD,), jnp.float32),
        ] + [pltpu.SemaphoreType.DMA] * 13,
    )(src3, dst3, yp, norm, zeros1, zeros2)


BLK = 1024


def _prep_body(dega_ref, degb_ref, x_ref, w1_ref, yp_ref, norm_ref):
    deg = dega_ref[...] + degb_ref[...]
    norm = jnp.where(deg > 0.0, lax.rsqrt(deg), 0.0)
    y = jnp.dot(x_ref[...], w1_ref[...], preferred_element_type=jnp.float32)
    yp_ref[...] = y * norm
    norm_ref[...] = norm


def _tc_prep(dega, degb, x_pad, W1):
    grid = N_PAD // BLK
    return pl.pallas_call(
        _prep_body,
        grid=(grid,),
        in_specs=[
            pl.BlockSpec((BLK, 1), lambda i: (i, 0)),
            pl.BlockSpec((BLK, 1), lambda i: (i, 0)),
            pl.BlockSpec((BLK, F), lambda i: (i, 0)),
            pl.BlockSpec((F, F), lambda i: (0, 0)),
        ],
        out_specs=[
            pl.BlockSpec((BLK, F), lambda i: (i, 0)),
            pl.BlockSpec((BLK, 1), lambda i: (i, 0)),
        ],
        out_shape=[
            jax.ShapeDtypeStruct((N_PAD, F), jnp.float32),
            jax.ShapeDtypeStruct((N_PAD, 1), jnp.float32),
        ],
    )(dega, degb, x_pad, W1)


def _final_body(rawa_ref, rawb_ref, ta_ref, tb_ref, norm_ref, b1_ref,
                w2_ref, b2_ref, out_ref, acc_ref):
    i = pl.program_id(0)
    norm = norm_ref[...]
    h1 = jnp.maximum(norm * (rawa_ref[...] + rawb_ref[...]) + b1_ref[...], 0.0)
    s = norm * (ta_ref[...] + tb_ref[...])
    gid = i * BLK + lax.broadcasted_iota(jnp.int32, (BLK, 1), 0)
    s = jnp.where(gid < N, s, 0.0)
    contrib = jnp.sum(s * h1, axis=0, keepdims=True)

    @pl.when(i == 0)
    def _():
        acc_ref[...] = jnp.zeros_like(acc_ref)

    acc_ref[...] += contrib

    @pl.when(i == pl.num_programs(0) - 1)
    def _():
        v = acc_ref[...] * (1.0 / N)
        out_ref[...] = (
            jnp.dot(v, w2_ref[...], preferred_element_type=jnp.float32)
            + b2_ref[...]
        )


def _tc_final(rawa, rawb, ta, tb, norm, b1r, W2p, b2p):
    grid = N_PAD // BLK
    return pl.pallas_call(
        _final_body,
        grid=(grid,),
        in_specs=[
            pl.BlockSpec((BLK, F), lambda i: (i, 0)),
            pl.BlockSpec((BLK, F), lambda i: (i, 0)),
            pl.BlockSpec((BLK, 1), lambda i: (i, 0)),
            pl.BlockSpec((BLK, 1), lambda i: (i, 0)),
            pl.BlockSpec((BLK, 1), lambda i: (i, 0)),
            pl.BlockSpec((1, F), lambda i: (0, 0)),
            pl.BlockSpec((F, F), lambda i: (0, 0)),
            pl.BlockSpec((1, F), lambda i: (0, 0)),
        ],
        out_specs=pl.BlockSpec((1, F), lambda i: (0, 0)),
        out_shape=jax.ShapeDtypeStruct((1, F), jnp.float32),
        scratch_shapes=[pltpu.VMEM((1, F), jnp.float32)],
    )(rawa, rawb, ta, tb, norm, b1r, W2p, b2p)


def kernel(x, edge_index, W1, b1, W2, b2):
    src = edge_index[0].astype(jnp.int32)
    dst = edge_index[1].astype(jnp.int32)
    n_pad_edges = EPT_PAD - EPT
    pad_block = DUMMY + (
        (jnp.arange(n_pad_edges, dtype=jnp.int32)[None, :]
         + 97 * jnp.arange(NW, dtype=jnp.int32)[:, None]) % (N_PAD - N))
    src3 = jnp.concatenate([src.reshape(NW, EPT), pad_block], axis=1)
    src3 = src3.reshape(NW, NCH_A, CH)
    dst3 = jnp.concatenate([dst.reshape(NW, EPT), pad_block], axis=1)
    dst3 = dst3.reshape(NW, NCH_A, CH)
    x_pad = jnp.pad(x, ((0, N_PAD - N), (0, 0)))
    zeros1 = jnp.zeros((N_PAD,), jnp.float32)
    zeros2 = jnp.zeros((N_PAD, F), jnp.float32)

    deg = _sc_degree(dst3, zeros1)
    dega = deg[0].reshape(N_PAD, 1)
    degb = deg[1].reshape(N_PAD, 1)

    yp, norm_col = _tc_prep(dega, degb, x_pad, W1)
    norm_flat = norm_col.reshape(N_PAD)

    raw, t = _sc_aggregate(src3, dst3, yp, norm_flat, zeros1, zeros2)

    W2p = jnp.pad(W2, ((0, 0), (0, F - W2.shape[1])))
    b2p = jnp.pad(b2, (0, F - b2.shape[0])).reshape(1, F)
    out = _tc_final(raw[0], raw[1], t[0].reshape(N_PAD, 1),
                    t[1].reshape(N_PAD, 1), norm_col, b1.reshape(1, F),
                    W2p, b2p)
    return out[0, :NUM_OUT]


# speedup vs baseline: 1.2808x; 1.0833x over previous
import functools

import jax
import jax.numpy as jnp
from jax import lax
from jax.experimental import pallas as pl
from jax.experimental.pallas import tpu as pltpu
from jax.experimental.pallas import tpu_sc as plsc

N = 10000
E = 320000
F = 128
NUM_OUT = 16
NC, NS = 2, 16
NW = NC * NS
EPT = E // NW
CH = 96
NCH_E = 105
NCH_A = NCH_E + 4
EPT_PAD = NCH_A * CH
IRING = 8
DUMMY = N
N_PAD = 10240
SLAB = N_PAD // NS


def _sc_mesh():
    return plsc.VectorSubcoreMesh(
        core_axis_name="c", subcore_axis_name="s", num_cores=NC, num_subcores=NS
    )


def _deg_body(dst_hbm, zeros1_hbm, deg_hbm, dstv, onesv, degsh, sems):
    cid = lax.axis_index("c")
    sid = lax.axis_index("s")
    wid = cid * NS + sid
    pltpu.sync_copy(zeros1_hbm, degsh.at[pl.ds(sid * SLAB, SLAB)])
    pltpu.sync_copy(dst_hbm.at[wid], dstv)
    for i in range(CH // 16):
        onesv[pl.ds(i * 16, 16)] = jnp.ones((16,), jnp.float32)
    plsc.subcore_barrier()

    def step(j, carry):
        pltpu.async_copy(onesv, degsh.at[dstv.at[j]], sems, add=True)
        return carry

    lax.fori_loop(0, NCH_E, step, 0)

    def drain(j, carry):
        pltpu.make_async_copy(onesv, degsh.at[dstv.at[0]], sems).wait()
        return carry

    lax.fori_loop(0, NCH_E, drain, 0)
    plsc.subcore_barrier()
    pltpu.sync_copy(degsh.at[pl.ds(sid * SLAB, SLAB)],
                    deg_hbm.at[cid, pl.ds(sid * SLAB, SLAB)])


def _sc_degree(dst3, zeros1):
    return pl.kernel(
        _deg_body,
        out_type=jax.ShapeDtypeStruct((NC, N_PAD), jnp.float32),
        mesh=_sc_mesh(),
        scratch_types=[
            pltpu.VMEM((NCH_A, CH), jnp.int32),
            pltpu.VMEM((CH,), jnp.float32),
            pltpu.VMEM_SHARED((N_PAD,), jnp.float32),
            pltpu.SemaphoreType.DMA,
        ],
    )(dst3, zeros1)


def _agg_body(src_hbm, dst_hbm, yp_hbm, norm_hbm, zeros1_hbm, zeros2_hbm,
              raw_hbm, t_hbm, srcv, dstv, rows0, rows1, rows2, nb0, nb1, nb2,
              rawsh, tsh, semi, semr0, semr1, semr2, semng0, semng1, semng2,
              semsr0, semsr1, semsr2, semsn0, semsn1, semsn2):
    cid = lax.axis_index("c")
    sid = lax.axis_index("s")
    wid = cid * NS + sid
    rows = (rows0, rows1, rows2)
    nb = (nb0, nb1, nb2)
    semr = (semr0, semr1, semr2)
    semng = (semng0, semng1, semng2)
    semsr = (semsr0, semsr1, semsr2)
    semsn = (semsn0, semsn1, semsn2)
    pltpu.sync_copy(zeros2_hbm, rawsh.at[pl.ds(sid * SLAB, SLAB)])
    pltpu.sync_copy(zeros1_hbm, tsh.at[pl.ds(sid * SLAB, SLAB)])
    pltpu.sync_copy(src_hbm.at[wid, 0], srcv.at[0])
    pltpu.sync_copy(dst_hbm.at[wid, 0], dstv.at[0])
    pltpu.sync_copy(src_hbm.at[wid, 1], srcv.at[1])
    pltpu.sync_copy(dst_hbm.at[wid, 1], dstv.at[1])
    pltpu.async_copy(src_hbm.at[wid, 2], srcv.at[2], semi)
    pltpu.async_copy(dst_hbm.at[wid, 2], dstv.at[2], semi)
    pltpu.async_copy(src_hbm.at[wid, 3], srcv.at[3], semi)
    pltpu.async_copy(dst_hbm.at[wid, 3], dstv.at[3], semi)
    plsc.subcore_barrier()
    pltpu.async_copy(yp_hbm.at[srcv.at[0]], rows0, semr0)
    pltpu.async_copy(norm_hbm.at[dstv.at[0]], nb0, semng0)
    pltpu.async_copy(yp_hbm.at[srcv.at[1]], rows1, semr1)
    pltpu.async_copy(norm_hbm.at[dstv.at[1]], nb1, semng1)

    def step(t_it, carry):
        for k in range(3):
            j = 3 * t_it + k
            km = (k + 2) % 3
            sj = lax.rem(j, IRING)
            s2 = lax.rem(j + 2, IRING)
            s4 = lax.rem(j + 4, IRING)
            pltpu.make_async_copy(yp_hbm.at[srcv.at[sj]], rows[k],
                                  semr[k]).wait()
            pltpu.async_copy(rows[k], rawsh.at[dstv.at[sj]], semsr[k],
                             add=True)
            pltpu.make_async_copy(norm_hbm.at[dstv.at[sj]], nb[k],
                                  semng[k]).wait()
            pltpu.async_copy(nb[k], tsh.at[srcv.at[sj]], semsn[k], add=True)

            @pl.when(j > 0)
            def _():
                pltpu.make_async_copy(rows[km], rawsh.at[dstv.at[sj]],
                                      semsr[km]).wait()
                pltpu.make_async_copy(nb[km], tsh.at[srcv.at[sj]],
                                      semsn[km]).wait()

            pltpu.async_copy(src_hbm.at[wid, j + 4], srcv.at[s4], semi)
            pltpu.async_copy(dst_hbm.at[wid, j + 4], dstv.at[s4], semi)
            for _ in range(2):
                pltpu.make_async_copy(src_hbm.at[wid, 0], srcv.at[s2],
                                      semi).wait()
            pltpu.async_copy(yp_hbm.at[srcv.at[s2]], rows[km], semr[km])
            pltpu.async_copy(norm_hbm.at[dstv.at[s2]], nb[km], semng[km])
        return carry

    lax.fori_loop(0, NCH_E // 3, step, 0)
    pltpu.make_async_copy(rows2, rawsh.at[dstv.at[0]], semsr2).wait()
    pltpu.make_async_copy(nb2, tsh.at[srcv.at[0]], semsn2).wait()
    pltpu.make_async_copy(yp_hbm.at[srcv.at[0]], rows0, semr0).wait()
    pltpu.make_async_copy(norm_hbm.at[dstv.at[0]], nb0, semng0).wait()
    pltpu.make_async_copy(yp_hbm.at[srcv.at[1]], rows1, semr1).wait()
    pltpu.make_async_copy(norm_hbm.at[dstv.at[1]], nb1, semng1).wait()
    for _ in range(4):
        pltpu.make_async_copy(src_hbm.at[wid, 0], srcv.at[0], semi).wait()
    plsc.subcore_barrier()
    pltpu.sync_copy(rawsh.at[pl.ds(sid * SLAB, SLAB)],
                    raw_hbm.at[cid, pl.ds(sid * SLAB, SLAB)])
    pltpu.sync_copy(tsh.at[pl.ds(sid * SLAB, SLAB)],
                    t_hbm.at[cid, pl.ds(sid * SLAB, SLAB)])


def _sc_aggregate(src3, dst3, yp, norm, zeros1, zeros2):
    return pl.kernel(
        _agg_body,
        out_type=(
            jax.ShapeDtypeStruct((NC, N_PAD, F), jnp.float32),
            jax.ShapeDtypeStruct((NC, N_PAD), jnp.float32),
        ),
        mesh=_sc_mesh(),
        scratch_types=[
            pltpu.VMEM((IRING, CH), jnp.int32),
            pltpu.VMEM((IRING, CH), jnp.int32),
            pltpu.VMEM((CH, F), jnp.float32),
            pltpu.VMEM((CH, F), jnp.float32),
            pltpu.VMEM((CH, F), jnp.float32),
            pltpu.VMEM((CH,), jnp.float32),
            pltpu.VMEM((CH,), jnp.float32),
            pltpu.VMEM((CH,), jnp.float32),
            pltpu.VMEM_SHARED((N_PAD, F), jnp.float32),
            pltpu.VMEM_SHARED((N_PAD,), jnp.float32),
        ] + [pltpu.SemaphoreType.DMA] * 13,
    )(src3, dst3, yp, norm, zeros1, zeros2)


BLK = 1024


def _prep_body(dega_ref, degb_ref, x_ref, w1_ref, yp_ref, norm_ref):
    deg = dega_ref[...] + degb_ref[...]
    norm = jnp.where(deg > 0.0, lax.rsqrt(deg), 0.0)
    y = jnp.dot(x_ref[...], w1_ref[...], preferred_element_type=jnp.float32)
    yp_ref[...] = y * norm
    norm_ref[...] = norm


def _tc_prep(dega, degb, x_pad, W1):
    grid = N_PAD // BLK
    return pl.pallas_call(
        _prep_body,
        grid=(grid,),
        in_specs=[
            pl.BlockSpec((BLK, 1), lambda i: (i, 0)),
            pl.BlockSpec((BLK, 1), lambda i: (i, 0)),
            pl.BlockSpec((BLK, F), lambda i: (i, 0)),
            pl.BlockSpec((F, F), lambda i: (0, 0)),
        ],
        out_specs=[
            pl.BlockSpec((BLK, F), lambda i: (i, 0)),
            pl.BlockSpec((BLK, 1), lambda i: (i, 0)),
        ],
        out_shape=[
            jax.ShapeDtypeStruct((N_PAD, F), jnp.float32),
            jax.ShapeDtypeStruct((N_PAD, 1), jnp.float32),
        ],
    )(dega, degb, x_pad, W1)


def _final_body(rawa_ref, rawb_ref, ta_ref, tb_ref, norm_ref, b1_ref,
                w2_ref, b2_ref, out_ref, acc_ref):
    i = pl.program_id(0)
    norm = norm_ref[...]
    h1 = jnp.maximum(norm * (rawa_ref[...] + rawb_ref[...]) + b1_ref[...], 0.0)
    s = norm * (ta_ref[...] + tb_ref[...])
    gid = i * BLK + lax.broadcasted_iota(jnp.int32, (BLK, 1), 0)
    s = jnp.where(gid < N, s, 0.0)
    contrib = jnp.sum(s * h1, axis=0, keepdims=True)

    @pl.when(i == 0)
    def _():
        acc_ref[...] = jnp.zeros_like(acc_ref)

    acc_ref[...] += contrib

    @pl.when(i == pl.num_programs(0) - 1)
    def _():
        v = acc_ref[...] * (1.0 / N)
        out_ref[...] = (
            jnp.dot(v, w2_ref[...], preferred_element_type=jnp.float32)
            + b2_ref[...]
        )


def _tc_final(rawa, rawb, ta, tb, norm, b1r, W2p, b2p):
    grid = N_PAD // BLK
    return pl.pallas_call(
        _final_body,
        grid=(grid,),
        in_specs=[
            pl.BlockSpec((BLK, F), lambda i: (i, 0)),
            pl.BlockSpec((BLK, F), lambda i: (i, 0)),
            pl.BlockSpec((BLK, 1), lambda i: (i, 0)),
            pl.BlockSpec((BLK, 1), lambda i: (i, 0)),
            pl.BlockSpec((BLK, 1), lambda i: (i, 0)),
            pl.BlockSpec((1, F), lambda i: (0, 0)),
            pl.BlockSpec((F, F), lambda i: (0, 0)),
            pl.BlockSpec((1, F), lambda i: (0, 0)),
        ],
        out_specs=pl.BlockSpec((1, F), lambda i: (0, 0)),
        out_shape=jax.ShapeDtypeStruct((1, F), jnp.float32),
        scratch_shapes=[pltpu.VMEM((1, F), jnp.float32)],
    )(rawa, rawb, ta, tb, norm, b1r, W2p, b2p)


def kernel(x, edge_index, W1, b1, W2, b2):
    src = edge_index[0].astype(jnp.int32)
    dst = edge_index[1].astype(jnp.int32)
    n_pad_edges = EPT_PAD - EPT
    pad_block = DUMMY + (
        (jnp.arange(n_pad_edges, dtype=jnp.int32)[None, :]
         + 97 * jnp.arange(NW, dtype=jnp.int32)[:, None]) % (N_PAD - N))
    src3 = jnp.concatenate([src.reshape(NW, EPT), pad_block], axis=1)
    src3 = src3.reshape(NW, NCH_A, CH)
    dst3 = jnp.concatenate([dst.reshape(NW, EPT), pad_block], axis=1)
    dst3 = dst3.reshape(NW, NCH_A, CH)
    x_pad = jnp.pad(x, ((0, N_PAD - N), (0, 0)))
    zeros1 = jnp.zeros((SLAB,), jnp.float32)
    zeros2 = jnp.zeros((SLAB, F), jnp.float32)

    deg = _sc_degree(dst3, zeros1)
    dega = deg[0].reshape(N_PAD, 1)
    degb = deg[1].reshape(N_PAD, 1)

    yp, norm_col = _tc_prep(dega, degb, x_pad, W1)
    norm_flat = norm_col.reshape(N_PAD)

    raw, t = _sc_aggregate(src3, dst3, yp, norm_flat, zeros1, zeros2)

    W2p = jnp.pad(W2, ((0, 0), (0, F - W2.shape[1])))
    b2p = jnp.pad(b2, (0, F - b2.shape[0])).reshape(1, F)
    out = _tc_final(raw[0], raw[1], t[0].reshape(N_PAD, 1),
                    t[1].reshape(N_PAD, 1), norm_col, b1.reshape(1, F),
                    W2p, b2p)
    return out[0, :NUM_OUT]
